# baseline (speedup 1.0000x reference)
import jax
import jax.numpy as jnp
from jax import lax
from jax.experimental import pallas as pl
from jax.experimental.pallas import tpu as pltpu
from jax.experimental.pallas import tpu_sc as plsc

N = 10000
E = 320000
D_IN = 128
H = 16
SLOPE = 0.01

NC = 2
NS = 16
NW = NC * NS
CH = 128
NCH = 80
ET = NCH * CH
EPAD = NW * ET - E
NROWS = N + 16
RPT = N // NS

def _leaky(v):
    return jnp.maximum(v, SLOPE * v)


def _proj_body(x_ref, wcat_ref, bl_ref, y_ref, r_ref):
    w = wcat_ref[...]
    for u in range(8):
        xu = x_ref[:, u, :]
        yr = lax.dot_general(xu, w, (((1,), (1,)), ((), ())),
                             preferred_element_type=jnp.float32)
        y_ref[:, H * u:H * u + H] = yr[:, :H]
        r_ref[:, H * u:H * u + H] = yr[:, H:] + bl_ref[...]


def _project(x, wcat, bl2):
    return pl.pallas_call(
        _proj_body,
        out_shape=[
            jax.ShapeDtypeStruct((N // 8, 128), jnp.float32),
            jax.ShapeDtypeStruct((N // 8, 128), jnp.float32),
        ],
    )(x, wcat, bl2)


NBUF = 8
NSUP = NCH // NBUF
NPAIR = NSUP // 2


def _sc_body(y_hbm, eiw_hbm, out_hbm,
             idx_v, rows_a, rows_b, stage_v, acc_sh, sem_g, sem_s):
    cid = lax.axis_index("c")
    sid = lax.axis_index("s")
    wid = cid * NS + sid

    idx_g = pltpu.async_copy(eiw_hbm.at[wid], idx_v, sem_g)

    zero = jnp.zeros((16,), jnp.float32)

    def zbody(i, carry):
        stage_v[i] = zero
        return carry

    lax.fori_loop(0, RPT, zbody, 0)
    pltpu.sync_copy(stage_v, acc_sh.at[pl.ds(sid * RPT, RPT)])
    idx_g.wait()
    plsc.subcore_barrier()

    def issue_gathers(s, rows):
        for b in range(NBUF):
            pltpu.async_copy(y_hbm.at[idx_v.at[s * NBUF + b, 0]], rows[b],
                             sem_g)

    def drain_gathers(s, rows):
        for b in range(NBUF):
            pltpu.make_async_copy(y_hbm.at[idx_v.at[s * NBUF + b, 0]],
                                  rows[b], sem_g).wait()

    def issue_scatters(s, rows):
        for b in range(NBUF):
            pltpu.async_copy(rows[b], acc_sh.at[idx_v.at[s * NBUF + b, 1]],
                             sem_s, add=True)

    def drain_scatters(s, rows):
        for b in range(NBUF):
            pltpu.make_async_copy(rows[b], acc_sh.at[idx_v.at[s * NBUF + b, 1]],
                                  sem_s).wait()

    rows_a = list(rows_a)
    rows_b = list(rows_b)
    issue_gathers(0, rows_a)

    def pair_body(sp, carry):
        s0 = 2 * sp
        s1 = s0 + 1
        drain_gathers(s0, rows_a)

        @pl.when(sp > 0)
        def _():
            drain_scatters(s1 - 2, rows_b)

        issue_scatters(s0, rows_a)
        issue_gathers(s1, rows_b)
        drain_gathers(s1, rows_b)
        drain_scatters(s0, rows_a)
        issue_scatters(s1, rows_b)

        @pl.when(sp < NPAIR - 1)
        def _():
            issue_gathers(s0 + 2, rows_a)

        return carry

    lax.fori_loop(0, NPAIR, pair_body, 0)
    drain_scatters(NSUP - 1, rows_b)
    plsc.subcore_barrier()

    pltpu.sync_copy(acc_sh.at[pl.ds(sid * RPT, RPT)], stage_v)
    pltpu.sync_copy(stage_v, out_hbm.at[cid, sid])


def _sc_segment_sum(y, eiw):
    mesh = plsc.VectorSubcoreMesh(core_axis_name="c", subcore_axis_name="s")
    f = pl.kernel(
        _sc_body,
        out_type=jax.ShapeDtypeStruct((NC, NS, RPT, H), jnp.float32),
        mesh=mesh,
        compiler_params=pltpu.CompilerParams(use_tc_tiling_on_sc=False),
        scratch_types=[
            pltpu.VMEM((NCH, 2, CH), jnp.int32),
            [pltpu.VMEM((CH, H), jnp.float32) for _ in range(NBUF)],
            [pltpu.VMEM((CH, H), jnp.float32) for _ in range(NBUF)],
            pltpu.VMEM((RPT, H), jnp.float32),
            pltpu.VMEM_SHARED((NROWS, H), jnp.float32),
            pltpu.SemaphoreType.DMA,
            pltpu.SemaphoreType.DMA,
        ],
    )
    return f(y, eiw)


def _mlp_body(acc_ref, r_ref, w1_ref, b1_ref, w2_ref, b2_ref, o_ref):
    h = acc_ref[0] + acc_ref[1] + r_ref[...]
    h = _leaky(h)
    h = lax.dot_general(h, w1_ref[...], (((1,), (0,)), ((), ())),
                        preferred_element_type=jnp.float32) + b1_ref[...]
    h = _leaky(h)
    o_ref[...] = lax.dot_general(h, w2_ref[...], (((1,), (0,)), ((), ())),
                                 preferred_element_type=jnp.float32) + b2_ref[...]


def _mlp(acc_pk, r_pk, w1k, b1k, w2k, b2k):
    return pl.pallas_call(
        _mlp_body,
        out_shape=jax.ShapeDtypeStruct((N // 8, 128), jnp.float32),
    )(acc_pk, r_pk, w1k, b1k, w2k, b2k)


def kernel(x, edge_index, Wl, bl, Wr, W1, b1, W2, b2):
    ei = edge_index.astype(jnp.int32)
    extra = jnp.stack([jnp.zeros((EPAD,), jnp.int32),
                       jnp.full((EPAD,), N, jnp.int32)])
    eiw = (jnp.concatenate([ei, extra], axis=1)
           .reshape(2, NW * NCH, CH).transpose(1, 0, 2)
           .reshape(NW, NCH, 2, CH))
    wcat = jnp.concatenate([Wl, Wr], axis=0)
    y_pk, r_pk = _project(x.reshape(N // 8, 8, D_IN), wcat, bl.reshape(1, H))
    acc = _sc_segment_sum(y_pk.reshape(N, H), eiw)
    acc_pk = acc.reshape(NC, N // 8, 128)
    eye8 = jnp.eye(8, dtype=jnp.float32)
    w1k = jnp.kron(eye8, W1.T)
    w2k = jnp.kron(eye8, W2.T)
    b1k = jnp.tile(b1, 8).reshape(1, 128)
    b2k = jnp.tile(b2, 8).reshape(1, 128)
    out_pk = _mlp(acc_pk, r_pk, w1k, b1k, w2k, b2k)
    return out_pk.reshape(N, H)


# speedup vs baseline: 1.5855x; 1.5855x over previous
import jax
import jax.numpy as jnp
from jax import lax
from jax.experimental import pallas as pl
from jax.experimental.pallas import tpu as pltpu
from jax.experimental.pallas import tpu_sc as plsc

N = 10000
E = 320000
D_IN = 128
H = 16
SLOPE = 0.01

NC = 2
NS = 16
NW = NC * NS
CH = 128
NCH = 80
ET = NCH * CH
EPAD = NW * ET - E
NTRASH = 2048
NROWS = N + NTRASH
RPT = N // NS

def _leaky(v):
    return jnp.maximum(v, SLOPE * v)


def _proj_body(x_ref, wcat_ref, bl_ref, y_ref, r_ref):
    w = wcat_ref[...]
    for u in range(8):
        xu = x_ref[:, u, :]
        yr = lax.dot_general(xu, w, (((1,), (1,)), ((), ())),
                             preferred_element_type=jnp.float32)
        y_ref[:, H * u:H * u + H] = yr[:, :H]
        r_ref[:, H * u:H * u + H] = yr[:, H:] + bl_ref[...]


def _project(x, wcat, bl2):
    return pl.pallas_call(
        _proj_body,
        out_shape=[
            jax.ShapeDtypeStruct((N // 8, 128), jnp.float32),
            jax.ShapeDtypeStruct((N // 8, 128), jnp.float32),
        ],
    )(x, wcat, bl2)


NBUF = 8
NSUP = NCH // NBUF
NPAIR = NSUP // 2


def _sc_body(y_hbm, eiw_hbm, out_hbm,
             idx_v, rows_a, rows_b, stage_v, acc_sh, sem_g, sem_s):
    cid = lax.axis_index("c")
    sid = lax.axis_index("s")
    wid = cid * NS + sid

    idx_g = pltpu.async_copy(eiw_hbm.at[wid], idx_v, sem_g)

    zero = jnp.zeros((16,), jnp.float32)

    def zbody(i, carry):
        stage_v[i] = zero
        return carry

    lax.fori_loop(0, RPT, zbody, 0)
    pltpu.sync_copy(stage_v, acc_sh.at[pl.ds(sid * RPT, RPT)])
    idx_g.wait()
    plsc.subcore_barrier()

    def issue_gathers(s, rows):
        for b in range(NBUF):
            pltpu.async_copy(y_hbm.at[idx_v.at[s * NBUF + b, 0]], rows[b],
                             sem_g)

    def drain_gathers(s, rows):
        for b in range(NBUF):
            pltpu.make_async_copy(y_hbm.at[idx_v.at[s * NBUF + b, 0]],
                                  rows[b], sem_g).wait()

    def issue_scatters(s, rows):
        for b in range(NBUF):
            pltpu.async_copy(rows[b], acc_sh.at[idx_v.at[s * NBUF + b, 1]],
                             sem_s, add=True)

    def drain_scatters(s, rows):
        for b in range(NBUF):
            pltpu.make_async_copy(rows[b], acc_sh.at[idx_v.at[s * NBUF + b, 1]],
                                  sem_s).wait()

    rows_a = list(rows_a)
    rows_b = list(rows_b)
    issue_gathers(0, rows_a)

    def pair_body(sp, carry):
        s0 = 2 * sp
        s1 = s0 + 1
        drain_gathers(s0, rows_a)

        @pl.when(sp > 0)
        def _():
            drain_scatters(s1 - 2, rows_b)

        issue_scatters(s0, rows_a)
        issue_gathers(s1, rows_b)
        drain_gathers(s1, rows_b)
        drain_scatters(s0, rows_a)
        issue_scatters(s1, rows_b)

        @pl.when(sp < NPAIR - 1)
        def _():
            issue_gathers(s0 + 2, rows_a)

        return carry

    lax.fori_loop(0, NPAIR, pair_body, 0)
    drain_scatters(NSUP - 1, rows_b)
    plsc.subcore_barrier()

    pltpu.sync_copy(acc_sh.at[pl.ds(sid * RPT, RPT)], stage_v)
    pltpu.sync_copy(stage_v, out_hbm.at[cid, sid])


def _sc_segment_sum(y, eiw):
    mesh = plsc.VectorSubcoreMesh(core_axis_name="c", subcore_axis_name="s")
    f = pl.kernel(
        _sc_body,
        out_type=jax.ShapeDtypeStruct((NC, NS, RPT, H), jnp.float32),
        mesh=mesh,
        compiler_params=pltpu.CompilerParams(use_tc_tiling_on_sc=False),
        scratch_types=[
            pltpu.VMEM((NCH, 2, CH), jnp.int32),
            [pltpu.VMEM((CH, H), jnp.float32) for _ in range(NBUF)],
            [pltpu.VMEM((CH, H), jnp.float32) for _ in range(NBUF)],
            pltpu.VMEM((RPT, H), jnp.float32),
            pltpu.VMEM_SHARED((NROWS, H), jnp.float32),
            pltpu.SemaphoreType.DMA,
            pltpu.SemaphoreType.DMA,
        ],
    )
    return f(y, eiw)


def _mlp_body(acc_ref, r_ref, w1_ref, b1_ref, w2_ref, b2_ref, o_ref):
    h = acc_ref[0] + acc_ref[1] + r_ref[...]
    h = _leaky(h)
    h = lax.dot_general(h, w1_ref[...], (((1,), (0,)), ((), ())),
                        preferred_element_type=jnp.float32) + b1_ref[...]
    h = _leaky(h)
    o_ref[...] = lax.dot_general(h, w2_ref[...], (((1,), (0,)), ((), ())),
                                 preferred_element_type=jnp.float32) + b2_ref[...]


def _mlp(acc_pk, r_pk, w1k, b1k, w2k, b2k):
    return pl.pallas_call(
        _mlp_body,
        out_shape=jax.ShapeDtypeStruct((N // 8, 128), jnp.float32),
    )(acc_pk, r_pk, w1k, b1k, w2k, b2k)


def kernel(x, edge_index, Wl, bl, Wr, W1, b1, W2, b2):
    ei = edge_index.astype(jnp.int32)
    pad_ids = jnp.arange(EPAD, dtype=jnp.int32)
    extra = jnp.stack([pad_ids % N, N + pad_ids % NTRASH])
    eiw = (jnp.concatenate([ei, extra], axis=1)
           .reshape(2, NW * NCH, CH).transpose(1, 0, 2)
           .reshape(NW, NCH, 2, CH))
    wcat = jnp.concatenate([Wl, Wr], axis=0)
    y_pk, r_pk = _project(x.reshape(N // 8, 8, D_IN), wcat, bl.reshape(1, H))
    acc = _sc_segment_sum(y_pk.reshape(N, H), eiw)
    acc_pk = acc.reshape(NC, N // 8, 128)
    eye8 = jnp.eye(8, dtype=jnp.float32)
    w1k = jnp.kron(eye8, W1.T)
    w2k = jnp.kron(eye8, W2.T)
    b1k = jnp.tile(b1, 8).reshape(1, 128)
    b2k = jnp.tile(b2, 8).reshape(1, 128)
    out_pk = _mlp(acc_pk, r_pk, w1k, b1k, w2k, b2k)
    return out_pk.reshape(N, H)


# speedup vs baseline: 1.6709x; 1.0538x over previous
import jax
import jax.numpy as jnp
import numpy as np
from jax import lax
from jax.experimental import pallas as pl
from jax.experimental.pallas import tpu as pltpu
from jax.experimental.pallas import tpu_sc as plsc

N = 10000
E = 320000
D_IN = 128
H = 16
SLOPE = 0.01

NC = 2
NS = 16
NW = NC * NS
CH = 128
NCH = 80
ET = NCH * CH
EPAD = NW * ET - E
NTRASH = 2048
NROWS = N + NTRASH
RPT = N // NS

_PAD_IDS = np.arange(EPAD, dtype=np.int32)
_EXTRA = np.stack([_PAD_IDS % N, N + _PAD_IDS % NTRASH]).astype(np.int32)


def _leaky(v):
    return jnp.maximum(v, SLOPE * v)


def _proj_body(x_ref, wcat_ref, bl_ref, y_ref, r_ref):
    w = wcat_ref[...]
    for u in range(8):
        xu = x_ref[:, u, :]
        yr = lax.dot_general(xu, w, (((1,), (1,)), ((), ())),
                             preferred_element_type=jnp.float32)
        y_ref[:, H * u:H * u + H] = yr[:, :H]
        r_ref[:, H * u:H * u + H] = yr[:, H:] + bl_ref[...]


def _project(x, wcat, bl2):
    return pl.pallas_call(
        _proj_body,
        out_shape=[
            jax.ShapeDtypeStruct((N // 8, 128), jnp.float32),
            jax.ShapeDtypeStruct((N // 8, 128), jnp.float32),
        ],
    )(x, wcat, bl2)


NBUF = 20
NSUP = NCH // NBUF
NPAIR = NSUP // 2


def _sc_body(y_hbm, eiw_hbm, out_hbm,
             idx_v, rows_a, rows_b, stage_v, acc_sh, sem_g, sem_s):
    cid = lax.axis_index("c")
    sid = lax.axis_index("s")
    wid = cid * NS + sid

    idx_g = pltpu.async_copy(eiw_hbm.at[wid], idx_v, sem_g)

    zero = jnp.zeros((16,), jnp.float32)

    def zbody(i, carry):
        stage_v[i] = zero
        return carry

    lax.fori_loop(0, RPT, zbody, 0)
    idx_g.wait()

    def issue_gathers(s, rows):
        for b in range(NBUF):
            pltpu.async_copy(y_hbm.at[idx_v.at[s * NBUF + b, 0]], rows[b],
                             sem_g)

    def drain_gathers(s, rows):
        for b in range(NBUF):
            pltpu.make_async_copy(y_hbm.at[idx_v.at[s * NBUF + b, 0]],
                                  rows[b], sem_g).wait()

    def issue_scatters(s, rows):
        for b in range(NBUF):
            pltpu.async_copy(rows[b], acc_sh.at[idx_v.at[s * NBUF + b, 1]],
                             sem_s, add=True)

    def drain_scatters(s, rows):
        for b in range(NBUF):
            pltpu.make_async_copy(rows[b], acc_sh.at[idx_v.at[s * NBUF + b, 1]],
                                  sem_s).wait()

    rows_a = list(rows_a)
    rows_b = list(rows_b)
    issue_gathers(0, rows_a)
    pltpu.sync_copy(stage_v, acc_sh.at[pl.ds(sid * RPT, RPT)])
    plsc.subcore_barrier()

    def pair_body(sp, carry):
        s0 = 2 * sp
        s1 = s0 + 1
        drain_gathers(s0, rows_a)

        @pl.when(sp > 0)
        def _():
            drain_scatters(s1 - 2, rows_b)

        issue_scatters(s0, rows_a)
        issue_gathers(s1, rows_b)
        drain_gathers(s1, rows_b)
        drain_scatters(s0, rows_a)
        issue_scatters(s1, rows_b)

        @pl.when(sp < NPAIR - 1)
        def _():
            issue_gathers(s0 + 2, rows_a)

        return carry

    lax.fori_loop(0, NPAIR, pair_body, 0)
    drain_scatters(NSUP - 1, rows_b)
    plsc.subcore_barrier()

    pltpu.sync_copy(acc_sh.at[pl.ds(sid * RPT, RPT)], stage_v)
    pltpu.sync_copy(stage_v, out_hbm.at[cid, sid])


def _sc_segment_sum(y, eiw):
    mesh = plsc.VectorSubcoreMesh(core_axis_name="c", subcore_axis_name="s")
    f = pl.kernel(
        _sc_body,
        out_type=jax.ShapeDtypeStruct((NC, NS, RPT, H), jnp.float32),
        mesh=mesh,
        compiler_params=pltpu.CompilerParams(use_tc_tiling_on_sc=False),
        scratch_types=[
            pltpu.VMEM((NCH, 2, CH), jnp.int32),
            [pltpu.VMEM((CH, H), jnp.float32) for _ in range(NBUF)],
            [pltpu.VMEM((CH, H), jnp.float32) for _ in range(NBUF)],
            pltpu.VMEM((RPT, H), jnp.float32),
            pltpu.VMEM_SHARED((NROWS, H), jnp.float32),
            pltpu.SemaphoreType.DMA,
            pltpu.SemaphoreType.DMA,
        ],
    )
    return f(y, eiw)


def _mlp_body(acc_ref, r_ref, w1_ref, b1_ref, w2_ref, b2_ref, o_ref):
    h = acc_ref[0] + acc_ref[1] + r_ref[...]
    h = _leaky(h)
    h = lax.dot_general(h, w1_ref[...], (((1,), (0,)), ((), ())),
                        preferred_element_type=jnp.float32) + b1_ref[...]
    h = _leaky(h)
    o_ref[...] = lax.dot_general(h, w2_ref[...], (((1,), (0,)), ((), ())),
                                 preferred_element_type=jnp.float32) + b2_ref[...]


def _mlp(acc_pk, r_pk, w1k, b1k, w2k, b2k):
    return pl.pallas_call(
        _mlp_body,
        out_shape=jax.ShapeDtypeStruct((N // 8, 128), jnp.float32),
    )(acc_pk, r_pk, w1k, b1k, w2k, b2k)


def kernel(x, edge_index, Wl, bl, Wr, W1, b1, W2, b2):
    ei = edge_index.astype(jnp.int32)
    extra = jnp.asarray(_EXTRA)
    eiw = (jnp.concatenate([ei, extra], axis=1)
           .reshape(2, NW * NCH, CH).transpose(1, 0, 2)
           .reshape(NW, NCH, 2, CH))
    wcat = jnp.concatenate([Wl, Wr], axis=0)
    y_pk, r_pk = _project(x.reshape(N // 8, 8, D_IN), wcat, bl.reshape(1, H))
    acc = _sc_segment_sum(y_pk.reshape(N, H), eiw)
    acc_pk = acc.reshape(NC, N // 8, 128)
    eye8 = jnp.eye(8, dtype=jnp.float32)
    w1k = jnp.kron(eye8, W1.T)
    w2k = jnp.kron(eye8, W2.T)
    b1k = jnp.tile(b1, 8).reshape(1, 128)
    b2k = jnp.tile(b2, 8).reshape(1, 128)
    out_pk = _mlp(acc_pk, r_pk, w1k, b1k, w2k, b2k)
    return out_pk.reshape(N, H)


# speedup vs baseline: 1.8142x; 1.0858x over previous
import jax
import jax.numpy as jnp
import numpy as np
from jax import lax
from jax.experimental import pallas as pl
from jax.experimental.pallas import tpu as pltpu
from jax.experimental.pallas import tpu_sc as plsc

N = 10000
E = 320000
D_IN = 128
H = 16
SLOPE = 0.01

NC = 2
NS = 16
NW = NC * NS
CH = 128
NCH = 80
ET = NCH * CH
EPAD = NW * ET - E
NTRASH = 2048
NROWS = N + NTRASH
RPT = N // NS

NCHR = E // CH
_PAD_IDS = np.arange(EPAD, dtype=np.int32)
_EXTRA = np.stack([(_PAD_IDS % N).reshape(EPAD // CH, CH),
                   (N + _PAD_IDS % NTRASH).reshape(EPAD // CH, CH)],
                  axis=1).astype(np.int32)


def _leaky(v):
    return jnp.maximum(v, SLOPE * v)


def _proj_body(x_ref, wcat_ref, bl_ref, y_ref, r_ref):
    w = wcat_ref[...]
    for u in range(8):
        xu = x_ref[:, u, :]
        yr = lax.dot_general(xu, w, (((1,), (1,)), ((), ())),
                             preferred_element_type=jnp.float32)
        y_ref[:, H * u:H * u + H] = yr[:, :H]
        r_ref[:, H * u:H * u + H] = yr[:, H:] + bl_ref[...]


def _project(x, wcat, bl2):
    return pl.pallas_call(
        _proj_body,
        out_shape=[
            jax.ShapeDtypeStruct((N // 8, 128), jnp.float32),
            jax.ShapeDtypeStruct((N // 8, 128), jnp.float32),
        ],
    )(x, wcat, bl2)


NBUF = 20
NSUP = NCH // NBUF
NPAIR = NSUP // 2


def _sc_body(y_hbm, eiw_hbm, extra_hbm, out_hbm,
             idx_v, rows_a, rows_b, stage_v, acc_sh, sem_g, sem_s):
    cid = lax.axis_index("c")
    sid = lax.axis_index("s")
    wid = cid * NS + sid

    last = wid == NW - 1
    nreal = NCHR - (NW - 1) * NCH

    @pl.when(jnp.logical_not(last))
    def _():
        pltpu.async_copy(eiw_hbm.at[pl.ds(wid * NCH, NCH)], idx_v, sem_g)

    @pl.when(last)
    def _():
        pltpu.async_copy(eiw_hbm.at[pl.ds((NW - 1) * NCH, nreal)],
                         idx_v.at[pl.ds(0, nreal)], sem_g)
        pltpu.async_copy(extra_hbm, idx_v.at[pl.ds(nreal, NCH - nreal)],
                         sem_g)

    zero = jnp.zeros((16,), jnp.float32)

    def zbody(i, carry):
        stage_v[i] = zero
        return carry

    lax.fori_loop(0, RPT, zbody, 0)

    @pl.when(jnp.logical_not(last))
    def _():
        pltpu.make_async_copy(eiw_hbm.at[pl.ds(wid * NCH, NCH)], idx_v,
                              sem_g).wait()

    @pl.when(last)
    def _():
        pltpu.make_async_copy(eiw_hbm.at[pl.ds((NW - 1) * NCH, nreal)],
                              idx_v.at[pl.ds(0, nreal)], sem_g).wait()
        pltpu.make_async_copy(extra_hbm,
                              idx_v.at[pl.ds(nreal, NCH - nreal)],
                              sem_g).wait()

    def issue_gathers(s, rows):
        for b in range(NBUF):
            pltpu.async_copy(y_hbm.at[idx_v.at[s * NBUF + b, 0]], rows[b],
                             sem_g)

    def drain_gathers(s, rows):
        for b in range(NBUF):
            pltpu.make_async_copy(y_hbm.at[idx_v.at[s * NBUF + b, 0]],
                                  rows[b], sem_g).wait()

    def issue_scatters(s, rows):
        for b in range(NBUF):
            pltpu.async_copy(rows[b], acc_sh.at[idx_v.at[s * NBUF + b, 1]],
                             sem_s, add=True)

    def drain_scatters(s, rows):
        for b in range(NBUF):
            pltpu.make_async_copy(rows[b], acc_sh.at[idx_v.at[s * NBUF + b, 1]],
                                  sem_s).wait()

    rows_a = list(rows_a)
    rows_b = list(rows_b)
    issue_gathers(0, rows_a)
    pltpu.sync_copy(stage_v, acc_sh.at[pl.ds(sid * RPT, RPT)])
    plsc.subcore_barrier()

    def pair_body(sp, carry):
        s0 = 2 * sp
        s1 = s0 + 1
        drain_gathers(s0, rows_a)

        @pl.when(sp > 0)
        def _():
            drain_scatters(s1 - 2, rows_b)

        issue_scatters(s0, rows_a)
        issue_gathers(s1, rows_b)
        drain_gathers(s1, rows_b)
        drain_scatters(s0, rows_a)
        issue_scatters(s1, rows_b)

        @pl.when(sp < NPAIR - 1)
        def _():
            issue_gathers(s0 + 2, rows_a)

        return carry

    lax.fori_loop(0, NPAIR, pair_body, 0)
    drain_scatters(NSUP - 1, rows_b)
    plsc.subcore_barrier()

    pltpu.sync_copy(acc_sh.at[pl.ds(sid * RPT, RPT)], stage_v)
    pltpu.sync_copy(stage_v, out_hbm.at[cid, sid])


def _sc_segment_sum(y, eiw, extra):
    mesh = plsc.VectorSubcoreMesh(core_axis_name="c", subcore_axis_name="s")
    f = pl.kernel(
        _sc_body,
        out_type=jax.ShapeDtypeStruct((NC, NS, RPT, H), jnp.float32),
        mesh=mesh,
        compiler_params=pltpu.CompilerParams(use_tc_tiling_on_sc=False),
        scratch_types=[
            pltpu.VMEM((NCH, 2, CH), jnp.int32),
            [pltpu.VMEM((CH, H), jnp.float32) for _ in range(NBUF)],
            [pltpu.VMEM((CH, H), jnp.float32) for _ in range(NBUF)],
            pltpu.VMEM((RPT, H), jnp.float32),
            pltpu.VMEM_SHARED((NROWS, H), jnp.float32),
            pltpu.SemaphoreType.DMA,
            pltpu.SemaphoreType.DMA,
        ],
    )
    return f(y, eiw, extra)


def _mlp_body(acc_ref, r_ref, w1_ref, b1_ref, w2_ref, b2_ref, o_ref):
    h = acc_ref[0] + acc_ref[1] + r_ref[...]
    h = _leaky(h)
    h = lax.dot_general(h, w1_ref[...], (((1,), (0,)), ((), ())),
                        preferred_element_type=jnp.float32) + b1_ref[...]
    h = _leaky(h)
    o_ref[...] = lax.dot_general(h, w2_ref[...], (((1,), (0,)), ((), ())),
                                 preferred_element_type=jnp.float32) + b2_ref[...]


def _mlp(acc_pk, r_pk, w1k, b1k, w2k, b2k):
    return pl.pallas_call(
        _mlp_body,
        out_shape=jax.ShapeDtypeStruct((N // 8, 128), jnp.float32),
    )(acc_pk, r_pk, w1k, b1k, w2k, b2k)


def kernel(x, edge_index, Wl, bl, Wr, W1, b1, W2, b2):
    ei = edge_index.astype(jnp.int32)
    eiw = ei.reshape(2, NCHR, CH).transpose(1, 0, 2)
    extra = jnp.asarray(_EXTRA)
    wcat = jnp.concatenate([Wl, Wr], axis=0)
    y_pk, r_pk = _project(x.reshape(N // 8, 8, D_IN), wcat, bl.reshape(1, H))
    acc = _sc_segment_sum(y_pk.reshape(N, H), eiw, extra)
    acc_pk = acc.reshape(NC, N // 8, 128)
    eye8 = jnp.eye(8, dtype=jnp.float32)
    w1k = jnp.kron(eye8, W1.T)
    w2k = jnp.kron(eye8, W2.T)
    b1k = jnp.tile(b1, 8).reshape(1, 128)
    b2k = jnp.tile(b2, 8).reshape(1, 128)
    out_pk = _mlp(acc_pk, r_pk, w1k, b1k, w2k, b2k)
    return out_pk.reshape(N, H)
